# Initial kernel scaffold; baseline (speedup 1.0000x reference)
import jax
import jax.numpy as jnp
from jax.experimental import pallas as pl


def kernel(X, A_q, A_h, Adj, W1, b1, W2, b2):
    raise NotImplementedError("write your pallas kernel here")


# speedup vs baseline: 1.0158x; 1.0158x over previous
import jax
import jax.numpy as jnp
from jax.experimental import pallas as pl
from jax.experimental.pallas import tpu as pltpu

N = 4096
H = 24
Z = 64
BK = 256
NB = N // BK
BR = 512
NR = N // BR


def _gcn_body(xs_ref, adj_ref, w1_ref, b1_ref, w2_ref, b2_ref,
              out_ref, acc_ref, y1_ref, adjb_ref):
    i = pl.program_id(0)

    @pl.when(i == 0)
    def _init():
        y1 = jnp.dot(xs_ref[...], w1_ref[...],
                     preferred_element_type=jnp.float32)
        y1_ref[...] = y1.astype(jnp.bfloat16)
        acc_ref[...] = jnp.zeros_like(acc_ref)

    off = pl.multiple_of(i * BK, BK)
    ab = adj_ref[...].astype(jnp.bfloat16)
    adjb_ref[:, pl.ds(off, BK)] = ab
    acc_ref[...] += jnp.dot(ab, y1_ref[pl.ds(off, BK), :],
                            preferred_element_type=jnp.float32)

    @pl.when(i == NB - 1)
    def _final():
        s1 = jnp.maximum(acc_ref[...] + b1_ref[...][None, :], 0.0)
        y2 = jnp.dot(s1, w2_ref[...], preferred_element_type=jnp.float32)
        y2b = y2.astype(jnp.bfloat16)
        b2v = b2_ref[...][None, :]

        def _row_block(r, y2b):
            roff = pl.multiple_of(r * BR, BR)
            out_ref[pl.ds(roff, BR), :] = b2v + jnp.dot(
                adjb_ref[pl.ds(roff, BR), :], y2b,
                preferred_element_type=jnp.float32)
            return y2b

        jax.lax.fori_loop(0, NR, _row_block, y2b)


def _gcn(xs, Adj, W1, b1, W2, b2, interpret=False):
    return pl.pallas_call(
        _gcn_body,
        grid=(NB,),
        in_specs=[
            pl.BlockSpec((N, H), lambda i: (0, 0)),
            pl.BlockSpec((N, BK), lambda i: (0, i)),
            pl.BlockSpec((H, Z), lambda i: (0, 0)),
            pl.BlockSpec((Z,), lambda i: (0,)),
            pl.BlockSpec((Z, H), lambda i: (0, 0)),
            pl.BlockSpec((H,), lambda i: (0,)),
        ],
        out_specs=pl.BlockSpec((N, H), lambda i: (0, 0)),
        out_shape=jax.ShapeDtypeStruct((N, H), jnp.float32),
        scratch_shapes=[
            pltpu.VMEM((N, Z), jnp.float32),
            pltpu.VMEM((N, Z), jnp.bfloat16),
            pltpu.VMEM((N, N), jnp.bfloat16),
        ],
        interpret=interpret,
    )(xs, Adj, W1, b1, W2, b2)


def kernel(X, A_q, A_h, Adj, W1, b1, W2, b2):
    xs = jnp.transpose(X[0])
    out = _gcn(xs, Adj, W1, b1, W2, b2)
    return jnp.transpose(out)[None]


# speedup vs baseline: 1.2016x; 1.1829x over previous
import jax
import jax.numpy as jnp
from jax.experimental import pallas as pl
from jax.experimental.pallas import tpu as pltpu

N = 4096
H = 24
Z = 64
BK = 256
NB = N // BK
BR = 512
NR = N // BR


def _gcn_body(x0_ref, adj_ref, w1_ref, b1_ref, w2_ref, b2_ref,
              out_ref, acc_ref, y1_ref, adjb_ref):
    i = pl.program_id(0)

    @pl.when(i == 0)
    def _init():
        y1 = jax.lax.dot_general(
            x0_ref[...], w1_ref[...],
            dimension_numbers=(((0,), (0,)), ((), ())),
            preferred_element_type=jnp.float32)
        y1_ref[...] = y1.astype(jnp.bfloat16)
        acc_ref[...] = jnp.zeros_like(acc_ref)

    off = pl.multiple_of(i * BK, BK)
    ab = adj_ref[...].astype(jnp.bfloat16)
    adjb_ref[:, pl.ds(off, BK)] = ab
    acc_ref[...] += jnp.dot(ab, y1_ref[pl.ds(off, BK), :],
                            preferred_element_type=jnp.float32)

    @pl.when(i == NB - 1)
    def _final():
        s1 = jnp.maximum(acc_ref[...] + b1_ref[...][None, :], 0.0)
        y2 = jnp.dot(s1, w2_ref[...], preferred_element_type=jnp.float32)
        y2b = y2.astype(jnp.bfloat16)
        b2v = b2_ref[...][:, None]

        def _row_block(r, y2b):
            roff = pl.multiple_of(r * BR, BR)
            blk = jnp.dot(adjb_ref[pl.ds(roff, BR), :], y2b,
                          preferred_element_type=jnp.float32)
            out_ref[:, pl.ds(roff, BR)] = jnp.transpose(blk) + b2v
            return y2b

        jax.lax.fori_loop(0, NR, _row_block, y2b)


def _gcn(x0, Adj, W1, b1, W2, b2, interpret=False):
    return pl.pallas_call(
        _gcn_body,
        grid=(NB,),
        in_specs=[
            pl.BlockSpec((H, N), lambda i: (0, 0)),
            pl.BlockSpec((N, BK), lambda i: (0, i)),
            pl.BlockSpec((H, Z), lambda i: (0, 0)),
            pl.BlockSpec((Z,), lambda i: (0,)),
            pl.BlockSpec((Z, H), lambda i: (0, 0)),
            pl.BlockSpec((H,), lambda i: (0,)),
        ],
        out_specs=pl.BlockSpec((H, N), lambda i: (0, 0)),
        out_shape=jax.ShapeDtypeStruct((H, N), jnp.float32),
        scratch_shapes=[
            pltpu.VMEM((N, Z), jnp.float32),
            pltpu.VMEM((N, Z), jnp.bfloat16),
            pltpu.VMEM((N, N), jnp.bfloat16),
        ],
        interpret=interpret,
    )(x0, Adj, W1, b1, W2, b2)


def kernel(X, A_q, A_h, Adj, W1, b1, W2, b2):
    out = _gcn(X[0], Adj, W1, b1, W2, b2)
    return out[None]


# speedup vs baseline: 1.2877x; 1.0716x over previous
import jax
import jax.numpy as jnp
from jax.experimental import pallas as pl
from jax.experimental.pallas import tpu as pltpu

N = 4096
H = 24
Z = 64
BM = 512
NB = N // BM
BR = 512
NR = N // BR


def _gcn_body(x0_ref, adj_ref, w1_ref, b1_ref, w2_ref, b2_ref,
              out_ref, y1_ref, y2_ref, adjb_ref):
    i = pl.program_id(0)

    @pl.when(i == 0)
    def _init():
        y1 = jax.lax.dot_general(
            x0_ref[...], w1_ref[...],
            dimension_numbers=(((0,), (0,)), ((), ())),
            preferred_element_type=jnp.float32)
        y1_ref[...] = y1.astype(jnp.bfloat16)

    off = pl.multiple_of(i * BM, BM)
    ab = adj_ref[...].astype(jnp.bfloat16)
    adjb_ref[pl.ds(off, BM), :] = ab
    h1 = jnp.dot(ab, y1_ref[...], preferred_element_type=jnp.float32)
    s1 = jnp.maximum(h1 + b1_ref[...][None, :], 0.0)
    y2_ref[pl.ds(off, BM), :] = jnp.dot(
        s1, w2_ref[...], preferred_element_type=jnp.float32
    ).astype(jnp.bfloat16)

    @pl.when(i == NB - 1)
    def _final():
        b2v = b2_ref[...][:, None]
        y2b = y2_ref[...]

        def _row_block(r, y2b):
            roff = pl.multiple_of(r * BR, BR)
            blk = jnp.dot(adjb_ref[pl.ds(roff, BR), :], y2b,
                          preferred_element_type=jnp.float32)
            out_ref[:, pl.ds(roff, BR)] = jnp.transpose(blk) + b2v
            return y2b

        jax.lax.fori_loop(0, NR, _row_block, y2b)


def _gcn(x0, Adj, W1, b1, W2, b2, interpret=False):
    return pl.pallas_call(
        _gcn_body,
        grid=(NB,),
        in_specs=[
            pl.BlockSpec((H, N), lambda i: (0, 0)),
            pl.BlockSpec((BM, N), lambda i: (i, 0)),
            pl.BlockSpec((H, Z), lambda i: (0, 0)),
            pl.BlockSpec((Z,), lambda i: (0,)),
            pl.BlockSpec((Z, H), lambda i: (0, 0)),
            pl.BlockSpec((H,), lambda i: (0,)),
        ],
        out_specs=pl.BlockSpec((H, N), lambda i: (0, 0)),
        out_shape=jax.ShapeDtypeStruct((H, N), jnp.float32),
        scratch_shapes=[
            pltpu.VMEM((N, Z), jnp.bfloat16),
            pltpu.VMEM((N, H), jnp.bfloat16),
            pltpu.VMEM((N, N), jnp.bfloat16),
        ],
        interpret=interpret,
    )(x0, Adj, W1, b1, W2, b2)


def kernel(X, A_q, A_h, Adj, W1, b1, W2, b2):
    out = _gcn(X[0], Adj, W1, b1, W2, b2)
    return out[None]


# speedup vs baseline: 1.3006x; 1.0101x over previous
import jax
import jax.numpy as jnp
from jax.experimental import pallas as pl
from jax.experimental.pallas import tpu as pltpu

N = 4096
H = 24
Z = 64
BM = 512
NB = N // BM
BR = 512
NR = N // BR


def _gcn_body(x0_ref, adj_ref, w1_ref, b1_ref, w2_ref, b2_ref,
              out_ref, y1_ref, y2_ref, adjb_ref):
    i = pl.program_id(0)

    @pl.when(i == 0)
    def _init():
        y1 = jax.lax.dot_general(
            x0_ref[...], w1_ref[...],
            dimension_numbers=(((0,), (0,)), ((), ())),
            preferred_element_type=jnp.float32)
        y1_ref[...] = y1.astype(jnp.bfloat16)

    off = pl.multiple_of(i * BM, BM)
    ab = adj_ref[...].astype(jnp.bfloat16)
    adjb_ref[pl.ds(off, BM), :] = ab
    h1 = jnp.dot(ab, y1_ref[...], preferred_element_type=jnp.float32)
    s1 = jnp.maximum(h1 + b1_ref[...][None, :], 0.0)
    y2_ref[pl.ds(off, BM), :] = jnp.dot(
        s1.astype(jnp.bfloat16), w2_ref[...].astype(jnp.bfloat16),
        preferred_element_type=jnp.float32,
    ).astype(jnp.bfloat16)

    @pl.when(i == NB - 1)
    def _final():
        b2v = b2_ref[...][:, None]
        y2b = y2_ref[...]

        def _row_block(r, y2b):
            roff = pl.multiple_of(r * BR, BR)
            blk = jnp.dot(adjb_ref[pl.ds(roff, BR), :], y2b,
                          preferred_element_type=jnp.float32)
            out_ref[:, pl.ds(roff, BR)] = jnp.transpose(blk) + b2v
            return y2b

        jax.lax.fori_loop(0, NR, _row_block, y2b)


def _gcn(x0, Adj, W1, b1, W2, b2, interpret=False):
    return pl.pallas_call(
        _gcn_body,
        grid=(NB,),
        in_specs=[
            pl.BlockSpec((H, N), lambda i: (0, 0)),
            pl.BlockSpec((BM, N), lambda i: (i, 0)),
            pl.BlockSpec((H, Z), lambda i: (0, 0)),
            pl.BlockSpec((Z,), lambda i: (0,)),
            pl.BlockSpec((Z, H), lambda i: (0, 0)),
            pl.BlockSpec((H,), lambda i: (0,)),
        ],
        out_specs=pl.BlockSpec((H, N), lambda i: (0, 0)),
        out_shape=jax.ShapeDtypeStruct((H, N), jnp.float32),
        scratch_shapes=[
            pltpu.VMEM((N, Z), jnp.bfloat16),
            pltpu.VMEM((N, H), jnp.bfloat16),
            pltpu.VMEM((N, N), jnp.bfloat16),
        ],
        interpret=interpret,
    )(x0, Adj, W1, b1, W2, b2)


def kernel(X, A_q, A_h, Adj, W1, b1, W2, b2):
    out = _gcn(X[0], Adj, W1, b1, W2, b2)
    return out[None]


# speedup vs baseline: 1.3590x; 1.0448x over previous
import jax
import jax.numpy as jnp
from jax.experimental import pallas as pl
from jax.experimental.pallas import tpu as pltpu

N = 4096
H = 24
Z = 64
BM = 512
NB = N // BM
BR = 512
NR = N // BR


def _gcn_body(x0_ref, adj_ref, w1_ref, b1_ref, w2_ref, b2_ref,
              out_ref, y1_ref, y2_ref, adjb_ref):
    i = pl.program_id(0)

    @pl.when(i == 0)
    def _init():
        y1 = jax.lax.dot_general(
            x0_ref[...], w1_ref[...],
            dimension_numbers=(((0,), (0,)), ((), ())),
            preferred_element_type=jnp.float32)
        y1_ref[...] = y1.astype(jnp.bfloat16)

    off = pl.multiple_of(i * BM, BM)
    ab = adj_ref[...].astype(jnp.bfloat16)
    adjb_ref[pl.ds(off, BM), :] = ab
    h1 = jnp.dot(ab, y1_ref[...], preferred_element_type=jnp.float32)
    s1 = jnp.maximum(h1 + b1_ref[...][None, :], 0.0)
    y2_ref[pl.ds(off, BM), :] = jnp.dot(
        s1.astype(jnp.bfloat16), w2_ref[...].astype(jnp.bfloat16),
        preferred_element_type=jnp.float32,
    ).astype(jnp.bfloat16)

    @pl.when(i == NB - 1)
    def _final():
        b2v = b2_ref[...][:, None]
        y2b = y2_ref[...]
        for r in range(NR):
            roff = r * BR
            blk = jnp.dot(adjb_ref[pl.ds(roff, BR), :], y2b,
                          preferred_element_type=jnp.float32)
            out_ref[:, pl.ds(roff, BR)] = jnp.transpose(blk) + b2v


def _gcn(x0, Adj, W1, b1, W2, b2, interpret=False):
    return pl.pallas_call(
        _gcn_body,
        grid=(NB,),
        in_specs=[
            pl.BlockSpec((H, N), lambda i: (0, 0)),
            pl.BlockSpec((BM, N), lambda i: (i, 0)),
            pl.BlockSpec((H, Z), lambda i: (0, 0)),
            pl.BlockSpec((Z,), lambda i: (0,)),
            pl.BlockSpec((Z, H), lambda i: (0, 0)),
            pl.BlockSpec((H,), lambda i: (0,)),
        ],
        out_specs=pl.BlockSpec((H, N), lambda i: (0, 0)),
        out_shape=jax.ShapeDtypeStruct((H, N), jnp.float32),
        scratch_shapes=[
            pltpu.VMEM((N, Z), jnp.bfloat16),
            pltpu.VMEM((N, H), jnp.bfloat16),
            pltpu.VMEM((N, N), jnp.bfloat16),
        ],
        interpret=interpret,
    )(x0, Adj, W1, b1, W2, b2)


def kernel(X, A_q, A_h, Adj, W1, b1, W2, b2):
    out = _gcn(X[0], Adj, W1, b1, W2, b2)
    return out[None]
